# baseline (speedup 1.0000x reference)
import jax
import jax.numpy as jnp
from jax import lax
from jax.experimental import pallas as pl
from jax.experimental.pallas import tpu as pltpu
from jax.experimental.pallas import tpu_sc as plsc

NC = 2
NS = 16
NW = NC * NS
LANES = 16
CHUNK = 128
NBUF = 16
NRING = 2 * NBUF
SWIN = 32


def _sc_edge_pass(n_pad, d, k_chunks, with_gather):
    mesh = plsc.VectorSubcoreMesh(core_axis_name="c", subcore_axis_name="s")
    rps = n_pad // NS
    prow = rps * d // 128

    def body(*refs):
        if with_gather:
            (hs_hbm, ei_hbm, out_hbm, src_v, dst_v, rows_v, acc, tbl,
             pk16, pk128, ld_sem, ld_sem2, ld_sem3), gsems = refs[:13], refs[13:]
        else:
            (ei_hbm, out_hbm, dst_v, rows_v, acc,
             pk16, pk128, ld_sem, sem) = refs
        c = lax.axis_index("c")
        s = lax.axis_index("s")
        wid = s * NC + c
        zbuf = rows_v.at[0] if with_gather else rows_v

        pltpu.async_copy(ei_hbm.at[1, pl.ds(wid * k_chunks, k_chunks)],
                         dst_v, ld_sem)
        if with_gather:
            pltpu.async_copy(ei_hbm.at[0, pl.ds(wid * k_chunks, k_chunks)],
                             src_v, ld_sem2)
            pltpu.async_copy(hs_hbm.at[pl.ds(s * prow, prow)], pk128,
                             ld_sem3)

        @pl.loop(0, CHUNK)
        def _(i):
            for j in range(d // LANES):
                zbuf.at[i, pl.ds(j * LANES, LANES)][...] = jnp.zeros(
                    (LANES,), jnp.float32)

        @pl.loop(0, rps // CHUNK)
        def _(t):
            pltpu.sync_copy(zbuf, acc.at[pl.ds(s * rps + t * CHUNK, CHUNK)])

        if not with_gather:
            @pl.loop(0, CHUNK)
            def _(i):
                for j in range(d // LANES):
                    rows_v.at[i, pl.ds(j * LANES, LANES)][...] = jnp.full(
                        (LANES,), 1.0, jnp.float32)

        pltpu.make_async_copy(
            ei_hbm.at[1, pl.ds(wid * k_chunks, k_chunks)], dst_v,
            ld_sem).wait()
        if with_gather:
            pltpu.make_async_copy(
                ei_hbm.at[0, pl.ds(wid * k_chunks, k_chunks)], src_v,
                ld_sem2).wait()
            pltpu.make_async_copy(
                hs_hbm.at[pl.ds(s * prow, prow)], pk128, ld_sem3).wait()

            @pl.loop(0, prow)
            def _(r):
                for cc in range(128 // LANES):
                    pk16.at[r * (128 // LANES) + cc][...] = (
                        pk128.at[r, pl.ds(cc * LANES, LANES)][...])

            pltpu.sync_copy(pk16, tbl.at[pl.ds(s * rps, rps)])
        plsc.subcore_barrier()

        if with_gather:
            for b in range(NBUF):
                pltpu.async_copy(tbl.at[src_v.at[b]], rows_v.at[b], gsems[b])

            @pl.loop(0, k_chunks, step=NBUF)
            def _(k):
                for b in range(NBUF):
                    pltpu.make_async_copy(
                        tbl.at[src_v.at[b]], rows_v.at[b], gsems[b]).wait()
                    pltpu.sync_copy(rows_v.at[b], acc.at[dst_v.at[k + b]],
                                    add=True)
                    nxt = k + b + NBUF

                    @pl.when(nxt < k_chunks)
                    def _():
                        pltpu.async_copy(
                            tbl.at[src_v.at[nxt]], rows_v.at[b], gsems[b])
        else:
            @pl.loop(0, k_chunks)
            def _(k):
                pltpu.async_copy(rows_v, acc.at[dst_v.at[k]], sem, add=True)

                @pl.when(k >= SWIN)
                def _():
                    pltpu.make_async_copy(
                        rows_v, acc.at[dst_v.at[k]], sem).wait()

            @pl.loop(0, SWIN)
            def _(k):
                pltpu.make_async_copy(rows_v, acc.at[dst_v.at[0]], sem).wait()

        plsc.subcore_barrier()
        pltpu.sync_copy(acc.at[pl.ds(s * rps, rps)], pk16)

        @pl.loop(0, prow)
        def _(r):
            for cc in range(128 // LANES):
                pk128.at[r, pl.ds(cc * LANES, LANES)][...] = (
                    pk16.at[r * (128 // LANES) + cc][...])

        pltpu.sync_copy(pk128, out_hbm.at[c, pl.ds(s * prow, prow)])

    if with_gather:
        scratch = [
            pltpu.VMEM((k_chunks, CHUNK), jnp.int32),
            pltpu.VMEM((k_chunks, CHUNK), jnp.int32),
            pltpu.VMEM((NBUF, CHUNK, d), jnp.float32),
            pltpu.VMEM_SHARED((n_pad, d), jnp.float32),
            pltpu.VMEM_SHARED((n_pad, d), jnp.float32),
            pltpu.VMEM((n_pad // NS, d), jnp.float32),
            pltpu.VMEM((n_pad * d // 128 // NS, 128), jnp.float32),
            pltpu.SemaphoreType.DMA,
            pltpu.SemaphoreType.DMA,
            pltpu.SemaphoreType.DMA,
        ] + [pltpu.SemaphoreType.DMA] * NBUF
    else:
        scratch = [
            pltpu.VMEM((k_chunks, CHUNK), jnp.int32),
            pltpu.VMEM((CHUNK, d), jnp.float32),
            pltpu.VMEM_SHARED((n_pad, d), jnp.float32),
            pltpu.VMEM((n_pad // NS, d), jnp.float32),
            pltpu.VMEM((n_pad * d // 128 // NS, 128), jnp.float32),
            pltpu.SemaphoreType.DMA,
            pltpu.SemaphoreType.DMA,
        ]

    return pl.kernel(
        body,
        out_type=jax.ShapeDtypeStruct((NC, n_pad * d // 128, 128),
                                       jnp.float32),
        mesh=mesh,
        scratch_types=scratch,
        compiler_params=pltpu.CompilerParams(use_tc_tiling_on_sc=False),
    )


def _tc_matmul_packed(xr, wbd, prows):
    nr = xr.shape[0]

    def mm(x_ref, w_ref, o_ref):
        h1p = jnp.dot(x_ref[...], w_ref[...],
                      preferred_element_type=jnp.float32)
        o_ref[...] = jnp.pad(h1p, ((0, prows - nr), (0, 0)))

    out = jax.ShapeDtypeStruct((prows, 128), jnp.float32)
    return pl.pallas_call(mm, out_shape=out)(xr, wbd)


def _tc_final(pre, w2, b2):
    n = pre.shape[0]
    d_out = w2.shape[1]

    def body(pre_ref, w2_ref, b2_ref, o_ref):
        z = jnp.dot(pre_ref[...], w2_ref[...],
                    preferred_element_type=jnp.float32) + b2_ref[...][None, :]
        m = jnp.max(z, axis=1, keepdims=True)
        zm = z - m
        lse = jnp.log(jnp.sum(jnp.exp(zm), axis=1, keepdims=True))
        o_ref[...] = zm - lse

    out = jax.ShapeDtypeStruct((n, d_out), jnp.float32)
    return pl.pallas_call(body, out_shape=out)(pre, w2, b2)


def kernel(x, edge_index, W1, b1, W2, b2):
    n, d_in = x.shape
    d_hid = W1.shape[1]
    e = edge_index.shape[1]

    n_pad = -(-(n + 1) // (NS * CHUNK)) * (NS * CHUNK)
    kc = -(-e // (NW * CHUNK))
    k_chunks = -(-kc // NRING) * NRING
    e_pad = k_chunks * NW * CHUNK

    ei = jnp.pad(edge_index, ((0, 0), (0, e_pad - e)), constant_values=n)
    ei = ei.reshape(2, NW * k_chunks, CHUNK)

    degp = _sc_edge_pass(n_pad, LANES, k_chunks, with_gather=False)(ei)
    xr = x.reshape(n // 8, 8 * d_in)
    w1bd = jnp.kron(jnp.eye(8, dtype=W1.dtype), W1)
    h1p = _tc_matmul_packed(xr, w1bd, n_pad * d_hid // 128)

    dis16 = lax.rsqrt(degp[0] + degp[1] + 1.0)
    h1s = h1p * dis16
    b1t = jnp.tile(b1, 128 // d_hid)[None, :]

    p1 = _sc_edge_pass(n_pad, d_hid, k_chunks, with_gather=True)(h1s, ei)
    g = jnp.maximum((p1[0] + p1[1] + h1s) * dis16 + b1t, 0.0) * dis16
    p2 = _sc_edge_pass(n_pad, d_hid, k_chunks, with_gather=True)(g, ei)
    prep = (p2[0] + p2[1] + g) * dis16
    pre = prep.reshape(n_pad, d_hid)[:n]
    return _tc_final(pre, W2, b2)


# speedup vs baseline: 1.9004x; 1.9004x over previous
import jax
import jax.numpy as jnp
from jax import lax
from jax.experimental import pallas as pl
from jax.experimental.pallas import tpu as pltpu
from jax.experimental.pallas import tpu_sc as plsc

NC = 2
NS = 16
NW = NC * NS
LANES = 16
CHUNK = 128
NBUF = 8
NRING = 2 * NBUF
SWIN = 16


def _sc_edge_pass(n_pad, d, k_chunks, with_gather):
    mesh = plsc.VectorSubcoreMesh(core_axis_name="c", subcore_axis_name="s")
    rps = n_pad // NS
    prow = rps * d // 128

    def body(*refs):
        if with_gather:
            (hs_hbm, ei_hbm, out_hbm, src_v, dst_v, rows_v, acc, tbl,
             pk16, pk128, ld_sem, ld_sem2, ld_sem3), gsems = refs[:13], refs[13:]
        else:
            (ei_hbm, out_hbm, dst_v, rows_v, acc,
             pk16, pk128, ld_sem, sem) = refs
        c = lax.axis_index("c")
        s = lax.axis_index("s")
        wid = s * NC + c
        zbuf = rows_v.at[0] if with_gather else rows_v

        pltpu.async_copy(ei_hbm.at[1, pl.ds(wid * k_chunks, k_chunks)],
                         dst_v, ld_sem)
        if with_gather:
            pltpu.async_copy(ei_hbm.at[0, pl.ds(wid * k_chunks, k_chunks)],
                             src_v, ld_sem2)
            pltpu.async_copy(hs_hbm.at[pl.ds(s * prow, prow)], pk128,
                             ld_sem3)

        @pl.loop(0, CHUNK)
        def _(i):
            for j in range(d // LANES):
                zbuf.at[i, pl.ds(j * LANES, LANES)][...] = jnp.zeros(
                    (LANES,), jnp.float32)

        @pl.loop(0, rps // CHUNK)
        def _(t):
            pltpu.sync_copy(zbuf, acc.at[pl.ds(s * rps + t * CHUNK, CHUNK)])

        if not with_gather:
            @pl.loop(0, CHUNK)
            def _(i):
                for j in range(d // LANES):
                    rows_v.at[i, pl.ds(j * LANES, LANES)][...] = jnp.full(
                        (LANES,), 1.0, jnp.float32)

        pltpu.make_async_copy(
            ei_hbm.at[1, pl.ds(wid * k_chunks, k_chunks)], dst_v,
            ld_sem).wait()
        if with_gather:
            pltpu.make_async_copy(
                ei_hbm.at[0, pl.ds(wid * k_chunks, k_chunks)], src_v,
                ld_sem2).wait()
            pltpu.make_async_copy(
                hs_hbm.at[pl.ds(s * prow, prow)], pk128, ld_sem3).wait()

            @pl.loop(0, prow)
            def _(r):
                for cc in range(128 // LANES):
                    pk16.at[r * (128 // LANES) + cc][...] = (
                        pk128.at[r, pl.ds(cc * LANES, LANES)][...])

            pltpu.sync_copy(pk16, tbl.at[pl.ds(s * rps, rps)])
        plsc.subcore_barrier()

        if with_gather:
            for b in range(NBUF):
                pltpu.async_copy(tbl.at[src_v.at[b]], rows_v.at[b], gsems[b])

            @pl.loop(0, k_chunks, step=NBUF)
            def _(k):
                for b in range(NBUF):
                    pltpu.make_async_copy(
                        tbl.at[src_v.at[b]], rows_v.at[b], gsems[b]).wait()
                    pltpu.sync_copy(rows_v.at[b], acc.at[dst_v.at[k + b]],
                                    add=True)
                    nxt = k + b + NBUF

                    @pl.when(nxt < k_chunks)
                    def _():
                        pltpu.async_copy(
                            tbl.at[src_v.at[nxt]], rows_v.at[b], gsems[b])
        else:
            @pl.loop(0, k_chunks)
            def _(k):
                pltpu.async_copy(rows_v, acc.at[dst_v.at[k]], sem, add=True)

                @pl.when(k >= SWIN)
                def _():
                    pltpu.make_async_copy(
                        rows_v, acc.at[dst_v.at[k]], sem).wait()

            @pl.loop(0, SWIN)
            def _(k):
                pltpu.make_async_copy(rows_v, acc.at[dst_v.at[0]], sem).wait()

        plsc.subcore_barrier()
        pltpu.sync_copy(acc.at[pl.ds(s * rps, rps)], pk16)

        @pl.loop(0, prow)
        def _(r):
            for cc in range(128 // LANES):
                pk128.at[r, pl.ds(cc * LANES, LANES)][...] = (
                    pk16.at[r * (128 // LANES) + cc][...])

        pltpu.sync_copy(pk128, out_hbm.at[c, pl.ds(s * prow, prow)])

    if with_gather:
        scratch = [
            pltpu.VMEM((k_chunks, CHUNK), jnp.int32),
            pltpu.VMEM((k_chunks, CHUNK), jnp.int32),
            pltpu.VMEM((NBUF, CHUNK, d), jnp.float32),
            pltpu.VMEM_SHARED((n_pad, d), jnp.float32),
            pltpu.VMEM_SHARED((n_pad, d), jnp.float32),
            pltpu.VMEM((n_pad // NS, d), jnp.float32),
            pltpu.VMEM((n_pad * d // 128 // NS, 128), jnp.float32),
            pltpu.SemaphoreType.DMA,
            pltpu.SemaphoreType.DMA,
            pltpu.SemaphoreType.DMA,
        ] + [pltpu.SemaphoreType.DMA] * NBUF
    else:
        scratch = [
            pltpu.VMEM((k_chunks, CHUNK), jnp.int32),
            pltpu.VMEM((CHUNK, d), jnp.float32),
            pltpu.VMEM_SHARED((n_pad, d), jnp.float32),
            pltpu.VMEM((n_pad // NS, d), jnp.float32),
            pltpu.VMEM((n_pad * d // 128 // NS, 128), jnp.float32),
            pltpu.SemaphoreType.DMA,
            pltpu.SemaphoreType.DMA,
        ]

    return pl.kernel(
        body,
        out_type=jax.ShapeDtypeStruct((NC, n_pad * d // 128, 128),
                                       jnp.float32),
        mesh=mesh,
        scratch_types=scratch,
        compiler_params=pltpu.CompilerParams(use_tc_tiling_on_sc=False),
    )


def _tc_matmul_packed(xr, wbd, prows):
    nr = xr.shape[0]

    def mm(x_ref, w_ref, o_ref):
        h1p = jnp.dot(x_ref[...], w_ref[...],
                      preferred_element_type=jnp.float32)
        o_ref[...] = jnp.pad(h1p, ((0, prows - nr), (0, 0)))

    out = jax.ShapeDtypeStruct((prows, 128), jnp.float32)
    return pl.pallas_call(mm, out_shape=out)(xr, wbd)


def _tc_final(pre, w2, b2):
    n = pre.shape[0]
    d_out = w2.shape[1]

    def body(pre_ref, w2_ref, b2_ref, o_ref):
        z = jnp.dot(pre_ref[...], w2_ref[...],
                    preferred_element_type=jnp.float32) + b2_ref[...][None, :]
        m = jnp.max(z, axis=1, keepdims=True)
        zm = z - m
        lse = jnp.log(jnp.sum(jnp.exp(zm), axis=1, keepdims=True))
        o_ref[...] = zm - lse

    out = jax.ShapeDtypeStruct((n, d_out), jnp.float32)
    return pl.pallas_call(body, out_shape=out)(pre, w2, b2)


def kernel(x, edge_index, W1, b1, W2, b2):
    n, d_in = x.shape
    d_hid = W1.shape[1]
    e = edge_index.shape[1]

    n_pad = -(-(n + 1) // (NS * CHUNK)) * (NS * CHUNK)
    kc = -(-e // (NW * CHUNK))
    k_chunks = -(-kc // NRING) * NRING
    e_pad = k_chunks * NW * CHUNK

    ei = jnp.pad(edge_index, ((0, 0), (0, e_pad - e)), constant_values=n)
    ei = ei.reshape(2, NW * k_chunks, CHUNK)

    degp = _sc_edge_pass(n_pad, LANES, k_chunks, with_gather=False)(ei)
    xr = x.reshape(n // 8, 8 * d_in)
    w1bd = jnp.kron(jnp.eye(8, dtype=W1.dtype), W1)
    h1p = _tc_matmul_packed(xr, w1bd, n_pad * d_hid // 128)

    dis16 = lax.rsqrt(degp[0] + degp[1] + 1.0)
    h1s = h1p * dis16
    b1t = jnp.tile(b1, 128 // d_hid)[None, :]

    p1 = _sc_edge_pass(n_pad, d_hid, k_chunks, with_gather=True)(h1s, ei)
    g = jnp.maximum((p1[0] + p1[1] + h1s) * dis16 + b1t, 0.0) * dis16
    p2 = _sc_edge_pass(n_pad, d_hid, k_chunks, with_gather=True)(g, ei)
    prep = (p2[0] + p2[1] + g) * dis16
    pre = prep.reshape(n_pad, d_hid)[:n]
    return _tc_final(pre, W2, b2)
